# baseline (speedup 1.0000x reference)
import functools

import jax
import jax.numpy as jnp
from jax import lax
from jax.experimental import pallas as pl
from jax.experimental.pallas import tpu as pltpu
from jax.experimental.pallas import tpu_sc as plsc

ENTRIES = 1024
DIM = 64
ROWS = 16 * 576
ROW_BLK = 512


def _argmin_body(a2_ref, b2_ref, flat_ref, cbt_ref, idx_ref):
    mm = lax.dot_general(
        flat_ref[...], cbt_ref[...],
        dimension_numbers=(((1,), (0,)), ((), ())),
        preferred_element_type=jnp.float32,
    )
    d = (a2_ref[...] + b2_ref[...]) - 2.0 * mm
    m = jnp.min(d, axis=1, keepdims=True)
    ii = lax.broadcasted_iota(jnp.int32, d.shape, 1)
    idx_ref[...] = jnp.min(jnp.where(d == m, ii, jnp.int32(ENTRIES)), axis=1)


def _nearest_idx(flat, codebook):
    a2 = jnp.sum(flat ** 2, axis=1, keepdims=True)
    b2 = jnp.sum(codebook ** 2, axis=1)[None, :]
    cbt = codebook.T
    grid = ROWS // ROW_BLK
    return pl.pallas_call(
        _argmin_body,
        grid=(grid,),
        in_specs=[
            pl.BlockSpec((ROW_BLK, 1), lambda i: (i, 0)),
            pl.BlockSpec((1, ENTRIES), lambda i: (0, 0)),
            pl.BlockSpec((ROW_BLK, DIM), lambda i: (i, 0)),
            pl.BlockSpec((DIM, ENTRIES), lambda i: (0, 0)),
        ],
        out_specs=pl.BlockSpec((ROW_BLK,), lambda i: (i,)),
        out_shape=jax.ShapeDtypeStruct((ROWS,), jnp.int32),
    )(a2, b2, flat, cbt)


@functools.cache
def _make_gather():
    nc, ns = 2, 16
    nw = nc * ns
    b_per_w = ROWS // nw
    chunk = 96
    n_chunks = b_per_w // chunk
    mesh = plsc.VectorSubcoreMesh(core_axis_name="c", subcore_axis_name="s")

    @functools.partial(
        pl.kernel, mesh=mesh,
        compiler_params=pltpu.CompilerParams(use_tc_tiling_on_sc=False),
        out_type=jax.ShapeDtypeStruct((ROWS, DIM), jnp.float32),
        scratch_types=[
            pltpu.VMEM((b_per_w,), jnp.int32),
            pltpu.VMEM((b_per_w, DIM), jnp.float32),
            pltpu.SemaphoreType.DMA,
        ],
    )
    def gather(table_hbm, idx_hbm, out_hbm, idx_v, rows_v, sem):
        wid = lax.axis_index("s") * nc + lax.axis_index("c")
        base = wid * b_per_w
        pltpu.sync_copy(idx_hbm.at[pl.ds(base, b_per_w)], idx_v)
        copies = [
            pltpu.async_copy(
                table_hbm.at[idx_v.at[pl.ds(k * chunk, chunk)]],
                rows_v.at[pl.ds(k * chunk, chunk)],
                sem,
            )
            for k in range(n_chunks)
        ]
        for c in copies:
            c.wait()
        pltpu.sync_copy(rows_v, out_hbm.at[pl.ds(base, b_per_w)])

    return gather


def kernel(z, codebook):
    B, T, D = z.shape
    flat = z.reshape(-1, D)
    idx = _nearest_idx(flat, codebook)
    z_q = _make_gather()(codebook, idx)
    return z_q.reshape(B, T, D), idx.reshape(B, T)


# speedup vs baseline: 1.1815x; 1.1815x over previous
import functools

import jax
import jax.numpy as jnp
from jax import lax
from jax.experimental import pallas as pl
from jax.experimental.pallas import tpu as pltpu
from jax.experimental.pallas import tpu_sc as plsc

ENTRIES = 1024
DIM = 64
ROWS = 16 * 576
ROW_BLK = 512
RT = 64


LANES = 128


SUB = 8


def _argmin_body(a2_ref, b2_ref, flat_ref, cb_ref, idx_ref):
    mmT = lax.dot_general(
        cb_ref[...], flat_ref[...],
        dimension_numbers=(((1,), (1,)), ((), ())),
        preferred_element_type=jnp.float32,
    )
    a2 = a2_ref[...]
    b2 = b2_ref[...]
    blk = mmT.shape[1]
    ii0 = lax.broadcasted_iota(jnp.int32, (SUB, LANES), 0)
    for ct in range(blk // LANES):
        clo, chi = ct * LANES, (ct + 1) * LANES
        a2c = a2[:, clo:chi]
        best_v = best_i = None
        for k in range(ENTRIES // SUB):
            rlo, rhi = k * SUB, (k + 1) * SUB
            d = (b2[rlo:rhi, :] + a2c) - 2.0 * mmT[rlo:rhi, clo:chi]
            if k == 0:
                best_v, best_i = d, ii0
            else:
                upd = d < best_v
                best_v = jnp.minimum(best_v, d)
                best_i = jnp.where(upd, ii0 + jnp.int32(k * SUB), best_i)
        m = jnp.min(best_v, axis=0, keepdims=True)
        idx_ref[clo:chi] = jnp.min(
            jnp.where(best_v == m, best_i, jnp.int32(ENTRIES)), axis=0)


def _nearest_idx(flat, codebook):
    a2 = jnp.sum(flat ** 2, axis=1)[None, :]
    b2 = jnp.broadcast_to(
        jnp.sum(codebook ** 2, axis=1)[:, None], (ENTRIES, LANES))
    grid = ROWS // ROW_BLK
    return pl.pallas_call(
        _argmin_body,
        grid=(grid,),
        in_specs=[
            pl.BlockSpec((1, ROW_BLK), lambda i: (0, i)),
            pl.BlockSpec((ENTRIES, LANES), lambda i: (0, 0)),
            pl.BlockSpec((ROW_BLK, DIM), lambda i: (i, 0)),
            pl.BlockSpec((ENTRIES, DIM), lambda i: (0, 0)),
        ],
        out_specs=pl.BlockSpec((ROW_BLK,), lambda i: (i,)),
        out_shape=jax.ShapeDtypeStruct((ROWS,), jnp.int32),
    )(a2, b2, flat, codebook)


@functools.cache
def _make_gather():
    nc, ns = 2, 16
    nw = nc * ns
    b_per_w = ROWS // nw
    chunk = 96
    n_chunks = b_per_w // chunk
    mesh = plsc.VectorSubcoreMesh(core_axis_name="c", subcore_axis_name="s")

    @functools.partial(
        pl.kernel, mesh=mesh,
        compiler_params=pltpu.CompilerParams(use_tc_tiling_on_sc=False),
        out_type=jax.ShapeDtypeStruct((ROWS, DIM), jnp.float32),
        scratch_types=[
            pltpu.VMEM((b_per_w,), jnp.int32),
            pltpu.VMEM((b_per_w, DIM), jnp.float32),
            pltpu.SemaphoreType.DMA,
            pltpu.SemaphoreType.DMA,
        ],
    )
    def gather(table_hbm, idx_hbm, out_hbm, idx_v, rows_v, gsem, wsem):
        wid = lax.axis_index("s") * nc + lax.axis_index("c")
        base = wid * b_per_w
        pltpu.sync_copy(idx_hbm.at[pl.ds(base, b_per_w)], idx_v)
        gathers = [
            pltpu.async_copy(
                table_hbm.at[idx_v.at[pl.ds(k * chunk, chunk)]],
                rows_v.at[pl.ds(k * chunk, chunk)],
                gsem,
            )
            for k in range(n_chunks)
        ]
        writes = []
        for k in range(n_chunks):
            gathers[k].wait()
            writes.append(pltpu.async_copy(
                rows_v.at[pl.ds(k * chunk, chunk)],
                out_hbm.at[pl.ds(base + k * chunk, chunk)],
                wsem,
            ))
        for w in writes:
            w.wait()

    return gather


def kernel(z, codebook):
    B, T, D = z.shape
    flat = z.reshape(-1, D)
    idx = _nearest_idx(flat, codebook)
    z_q = _make_gather()(codebook, idx)
    return z_q.reshape(B, T, D), idx.reshape(B, T)


# speedup vs baseline: 1.2788x; 1.0824x over previous
import functools

import jax
import jax.numpy as jnp
from jax import lax
from jax.experimental import pallas as pl
from jax.experimental.pallas import tpu as pltpu
from jax.experimental.pallas import tpu_sc as plsc

ENTRIES = 1024
DIM = 64
BATCH = 16
TOK = 576
ROWS = BATCH * TOK
LANES = 128
SUB = 8
_TILES = [(0, 128), (128, 128), (256, 128), (384, 128), (512, 64)]


def _argmin_body(a2_ref, b2t_ref, zt_ref, cb_ref, idx_ref):
    mmT = lax.dot_general(
        cb_ref[...], zt_ref[0],
        dimension_numbers=(((1,), (0,)), ((), ())),
        preferred_element_type=jnp.float32,
    )
    b2t = b2t_ref[...]
    for clo, w in _TILES:
        a2c = a2_ref[0, :, clo:clo + w]
        ii0 = lax.broadcasted_iota(jnp.int32, (SUB, w), 0)
        best_v = best_i = None
        for k in range(ENTRIES // SUB):
            bk = jnp.broadcast_to(b2t[:, k:k + 1], (SUB, w))
            d = (bk + a2c) - 2.0 * mmT[k * SUB:(k + 1) * SUB, clo:clo + w]
            if k == 0:
                best_v, best_i = d, ii0
            else:
                upd = d < best_v
                best_v = jnp.minimum(best_v, d)
                best_i = jnp.where(upd, ii0 + jnp.int32(k * SUB), best_i)
        m = jnp.min(best_v, axis=0, keepdims=True)
        idx_ref[0, 0, clo:clo + w] = jnp.min(
            jnp.where(best_v == m, best_i, jnp.int32(ENTRIES)), axis=0)


def _nearest_idx(z, codebook):
    flat = z.reshape(-1, DIM)
    a2 = jnp.sum(flat ** 2, axis=1).reshape(BATCH, 1, TOK)
    b2t = jnp.sum(codebook ** 2, axis=1).reshape(LANES, SUB).T
    zt = jnp.transpose(z, (0, 2, 1))
    idx3 = pl.pallas_call(
        _argmin_body,
        grid=(BATCH,),
        in_specs=[
            pl.BlockSpec((1, 1, TOK), lambda i: (i, 0, 0)),
            pl.BlockSpec((SUB, LANES), lambda i: (0, 0)),
            pl.BlockSpec((1, DIM, TOK), lambda i: (i, 0, 0)),
            pl.BlockSpec((ENTRIES, DIM), lambda i: (0, 0)),
        ],
        out_specs=pl.BlockSpec((1, 1, TOK), lambda i: (i, 0, 0)),
        out_shape=jax.ShapeDtypeStruct((BATCH, 1, TOK), jnp.int32),
    )(a2, b2t, zt, codebook)
    return idx3.reshape(ROWS)


@functools.cache
def _make_gather():
    nc, ns = 2, 16
    nw = nc * ns
    b_per_w = ROWS // nw
    chunk = 96
    n_chunks = b_per_w // chunk
    mesh = plsc.VectorSubcoreMesh(core_axis_name="c", subcore_axis_name="s")

    @functools.partial(
        pl.kernel, mesh=mesh,
        compiler_params=pltpu.CompilerParams(use_tc_tiling_on_sc=False),
        out_type=jax.ShapeDtypeStruct((ROWS, DIM), jnp.float32),
        scratch_types=[
            pltpu.VMEM((b_per_w,), jnp.int32),
            pltpu.VMEM((b_per_w, DIM), jnp.float32),
            pltpu.SemaphoreType.DMA,
            pltpu.SemaphoreType.DMA,
        ],
    )
    def gather(table_hbm, idx_hbm, out_hbm, idx_v, rows_v, gsem, wsem):
        wid = lax.axis_index("s") * nc + lax.axis_index("c")
        base = wid * b_per_w
        pltpu.sync_copy(idx_hbm.at[pl.ds(base, b_per_w)], idx_v)
        gathers = [
            pltpu.async_copy(
                table_hbm.at[idx_v.at[pl.ds(k * chunk, chunk)]],
                rows_v.at[pl.ds(k * chunk, chunk)],
                gsem,
            )
            for k in range(n_chunks)
        ]
        writes = []
        for k in range(n_chunks):
            gathers[k].wait()
            writes.append(pltpu.async_copy(
                rows_v.at[pl.ds(k * chunk, chunk)],
                out_hbm.at[pl.ds(base + k * chunk, chunk)],
                wsem,
            ))
        for w in writes:
            w.wait()

    return gather


def kernel(z, codebook):
    B, T, D = z.shape
    idx = _nearest_idx(z, codebook)
    z_q = _make_gather()(codebook, idx)
    return z_q.reshape(B, T, D), idx.reshape(B, T)


# speedup vs baseline: 1.3723x; 1.0731x over previous
import functools

import jax
import jax.numpy as jnp
from jax import lax
from jax.experimental import pallas as pl
from jax.experimental.pallas import tpu as pltpu
from jax.experimental.pallas import tpu_sc as plsc

ENTRIES = 1024
DIM = 64
BATCH = 16
TOK = 576
ROWS = BATCH * TOK
LANES = 128
SUB = 8
BPS = 4
_TILES = [(0, 128), (128, 128), (256, 128), (384, 128), (512, 64)]


def _argmin_body(a2_ref, b2t_ref, zt_ref, cb_ref, idx_ref):
    b2t = b2t_ref[...]
    for b in range(BPS):
        mmT = lax.dot_general(
            cb_ref[...], zt_ref[b],
            dimension_numbers=(((1,), (0,)), ((), ())),
            preferred_element_type=jnp.float32,
        )
        for clo, w in _TILES:
            a2c = a2_ref[b, :, clo:clo + w]
            ii0 = lax.broadcasted_iota(jnp.int32, (SUB, w), 0)
            best_v = best_i = None
            for k in range(ENTRIES // SUB):
                bk = jnp.broadcast_to(b2t[:, k:k + 1], (SUB, w))
                d = (bk + a2c) - 2.0 * mmT[k * SUB:(k + 1) * SUB, clo:clo + w]
                if k == 0:
                    best_v, best_i = d, ii0
                else:
                    upd = d < best_v
                    best_v = jnp.minimum(best_v, d)
                    best_i = jnp.where(upd, ii0 + jnp.int32(k * SUB), best_i)
            m = jnp.min(best_v, axis=0, keepdims=True)
            idx_ref[b, 0, clo:clo + w] = jnp.min(
                jnp.where(best_v == m, best_i, jnp.int32(ENTRIES)), axis=0)


def _nearest_idx(z, codebook):
    flat = z.reshape(-1, DIM)
    a2 = jnp.sum(flat ** 2, axis=1).reshape(BATCH, 1, TOK)
    b2t = jnp.sum(codebook ** 2, axis=1).reshape(LANES, SUB).T
    zt = jnp.transpose(z, (0, 2, 1))
    return pl.pallas_call(
        _argmin_body,
        grid=(BATCH // BPS,),
        in_specs=[
            pl.BlockSpec((BPS, 1, TOK), lambda i: (i, 0, 0)),
            pl.BlockSpec((SUB, LANES), lambda i: (0, 0)),
            pl.BlockSpec((BPS, DIM, TOK), lambda i: (i, 0, 0)),
            pl.BlockSpec((ENTRIES, DIM), lambda i: (0, 0)),
        ],
        out_specs=pl.BlockSpec((BPS, 1, TOK), lambda i: (i, 0, 0)),
        out_shape=jax.ShapeDtypeStruct((BATCH, 1, TOK), jnp.int32),
    )(a2, b2t, zt, codebook)


@functools.cache
def _make_gather():
    nc, ns = 2, 16
    nw = nc * ns
    b_per_w = ROWS // nw
    halves = nw // BATCH
    chunk = 96
    n_chunks = b_per_w // chunk
    mesh = plsc.VectorSubcoreMesh(core_axis_name="c", subcore_axis_name="s")

    @functools.partial(
        pl.kernel, mesh=mesh,
        compiler_params=pltpu.CompilerParams(use_tc_tiling_on_sc=False),
        out_type=jax.ShapeDtypeStruct((BATCH, TOK, DIM), jnp.float32),
        scratch_types=[
            pltpu.VMEM((b_per_w,), jnp.int32),
            pltpu.VMEM((b_per_w, DIM), jnp.float32),
            pltpu.SemaphoreType.DMA,
            pltpu.SemaphoreType.DMA,
        ],
    )
    def gather(table_hbm, idx_hbm, out_hbm, idx_v, rows_v, gsem, wsem):
        wid = lax.axis_index("s") * nc + lax.axis_index("c")
        b = wid // halves
        t0 = (wid % halves) * b_per_w
        pltpu.sync_copy(idx_hbm.at[b, 0, pl.ds(t0, b_per_w)], idx_v)
        gathers = [
            pltpu.async_copy(
                table_hbm.at[idx_v.at[pl.ds(k * chunk, chunk)]],
                rows_v.at[pl.ds(k * chunk, chunk)],
                gsem,
            )
            for k in range(n_chunks)
        ]
        writes = []
        for k in range(n_chunks):
            gathers[k].wait()
            writes.append(pltpu.async_copy(
                rows_v.at[pl.ds(k * chunk, chunk)],
                out_hbm.at[b, pl.ds(t0 + k * chunk, chunk)],
                wsem,
            ))
        for w in writes:
            w.wait()

    return gather


def kernel(z, codebook):
    B, T, D = z.shape
    idx3 = _nearest_idx(z, codebook)
    z_q = _make_gather()(codebook, idx3)
    return z_q, idx3.reshape(B, T)


# speedup vs baseline: 1.3933x; 1.0153x over previous
import functools

import jax
import jax.numpy as jnp
from jax import lax
from jax.experimental import pallas as pl
from jax.experimental.pallas import tpu as pltpu
from jax.experimental.pallas import tpu_sc as plsc

ENTRIES = 1024
DIM = 64
BATCH = 16
TOK = 576
ROWS = BATCH * TOK
LANES = 128
SUB = 8
BPS = 8
_TILES = [(0, 128), (128, 128), (256, 128), (384, 128), (512, 64)]


def _argmin_body(a2_ref, b2t_ref, zt_ref, cb_ref, idx_ref):
    b2t = b2t_ref[...]
    for b in range(BPS):
        mmT = lax.dot_general(
            cb_ref[...], zt_ref[b],
            dimension_numbers=(((1,), (0,)), ((), ())),
            preferred_element_type=jnp.float32,
        )
        for clo, w in _TILES:
            a2c = a2_ref[b:b + 1, clo:clo + w]
            ii0 = lax.broadcasted_iota(jnp.int32, (SUB, w), 0)
            best_v = best_i = None
            for k in range(ENTRIES // SUB):
                bk = jnp.broadcast_to(b2t[:, k:k + 1], (SUB, w))
                d = (bk + a2c) - 2.0 * mmT[k * SUB:(k + 1) * SUB, clo:clo + w]
                if k == 0:
                    best_v, best_i = d, ii0
                else:
                    upd = d < best_v
                    best_v = jnp.minimum(best_v, d)
                    best_i = jnp.where(upd, ii0 + jnp.int32(k * SUB), best_i)
            m = jnp.min(best_v, axis=0, keepdims=True)
            idx_ref[b, clo:clo + w] = jnp.min(
                jnp.where(best_v == m, best_i, jnp.int32(ENTRIES)), axis=0)


def _nearest_idx(z, codebook):
    flat = z.reshape(-1, DIM)
    a2 = jnp.sum(flat ** 2, axis=1).reshape(BATCH, TOK)
    b2t = jnp.sum(codebook ** 2, axis=1).reshape(LANES, SUB).T
    zt = jnp.transpose(z, (0, 2, 1))
    return pl.pallas_call(
        _argmin_body,
        grid=(BATCH // BPS,),
        in_specs=[
            pl.BlockSpec((BPS, TOK), lambda i: (i, 0)),
            pl.BlockSpec((SUB, LANES), lambda i: (0, 0)),
            pl.BlockSpec((BPS, DIM, TOK), lambda i: (i, 0, 0)),
            pl.BlockSpec((ENTRIES, DIM), lambda i: (0, 0)),
        ],
        out_specs=pl.BlockSpec((BPS, TOK), lambda i: (i, 0)),
        out_shape=jax.ShapeDtypeStruct((BATCH, TOK), jnp.int32),
    )(a2, b2t, zt, codebook)


@functools.cache
def _make_gather():
    nc, ns = 2, 16
    nw = nc * ns
    b_per_w = ROWS // nw
    halves = nw // BATCH
    chunk = 96
    n_chunks = b_per_w // chunk
    mesh = plsc.VectorSubcoreMesh(core_axis_name="c", subcore_axis_name="s")

    @functools.partial(
        pl.kernel, mesh=mesh,
        compiler_params=pltpu.CompilerParams(use_tc_tiling_on_sc=False),
        out_type=jax.ShapeDtypeStruct((BATCH, TOK, DIM), jnp.float32),
        scratch_types=[
            pltpu.VMEM((b_per_w,), jnp.int32),
            pltpu.VMEM((b_per_w, DIM), jnp.float32),
            pltpu.SemaphoreType.DMA,
            pltpu.SemaphoreType.DMA,
        ],
    )
    def gather(table_hbm, idx_hbm, out_hbm, idx_v, rows_v, gsem, wsem):
        wid = lax.axis_index("s") * nc + lax.axis_index("c")
        b = wid // halves
        t0 = (wid % halves) * b_per_w
        pltpu.sync_copy(idx_hbm.at[b, pl.ds(t0, b_per_w)], idx_v)
        gathers = [
            pltpu.async_copy(
                table_hbm.at[idx_v.at[pl.ds(k * chunk, chunk)]],
                rows_v.at[pl.ds(k * chunk, chunk)],
                gsem,
            )
            for k in range(n_chunks)
        ]
        writes = []
        for k in range(n_chunks):
            gathers[k].wait()
            writes.append(pltpu.async_copy(
                rows_v.at[pl.ds(k * chunk, chunk)],
                out_hbm.at[b, pl.ds(t0 + k * chunk, chunk)],
                wsem,
            ))
        for w in writes:
            w.wait()

    return gather


def kernel(z, codebook):
    idx = _nearest_idx(z, codebook)
    z_q = _make_gather()(codebook, idx)
    return z_q, idx
